# baseline (speedup 1.0000x reference)
import jax
import jax.numpy as jnp
from jax import lax
from jax.experimental import pallas as pl
from jax.experimental.pallas import tpu as pltpu
from jax.experimental.pallas import tpu_sc as plsc

N = 10000
E = 320000
D_IN = 128
D_H = 128
D_OUT = 64
HD1 = D_H // 2
HD2 = D_OUT // 2

NC, NS = 2, 16
NPAD = 10240
TPB = NPAD // NS
K = 128
EPAD = 327680
EPT = EPAD // NS
NBLK = EPT // K
DEGW = 16
ROWS_B = 640
ROWS_F = 1000


def _mesh():
    return plsc.VectorSubcoreMesh(
        core_axis_name="c", subcore_axis_name="s", num_cores=NC, num_subcores=NS
    )


def _deg_body(dst_hbm, ones_hbm, zeros_hbm, deg_out, idx_v, ones_v, acc):
    c = lax.axis_index("c")
    s = lax.axis_index("s")
    pltpu.sync_copy(zeros_hbm.at[pl.ds(s * TPB, TPB)], acc.at[pl.ds(s * TPB, TPB)])
    pltpu.sync_copy(dst_hbm.at[s], idx_v)
    pltpu.sync_copy(ones_hbm, ones_v)
    plsc.subcore_barrier()

    def body(j, carry):
        pltpu.sync_copy(ones_v, acc.at[idx_v.at[j]], add=True)
        return carry

    half = NBLK // 2
    lax.fori_loop(c * half, (c + 1) * half, body, 0)
    plsc.subcore_barrier()
    pltpu.sync_copy(acc.at[pl.ds(s * TPB, TPB)], deg_out.at[c, pl.ds(s * TPB, TPB)])


NBUFS = 5
PREF = 2


def _gs_body(ylo_hbm, yhi_hbm, src_hbm, dst_hbm, zeros_hbm, acc_out,
             srci, dsti, rows0, rows1, rows2, rows3, rows4, acc,
             g0, g1, g2, g3, g4, s0, s1, s2, s3, s4):
    c = lax.axis_index("c")
    s = lax.axis_index("s")
    rows = (rows0, rows1, rows2, rows3, rows4)
    gsem = (g0, g1, g2, g3, g4)
    ssem = (s0, s1, s2, s3, s4)
    pltpu.sync_copy(zeros_hbm.at[pl.ds(s * TPB, TPB)], acc.at[pl.ds(s * TPB, TPB)])
    pltpu.sync_copy(src_hbm.at[s], srci)
    pltpu.sync_copy(dst_hbm.at[s], dsti)
    plsc.subcore_barrier()

    def _gather(j, b):
        @pl.when(c == 0)
        def _():
            pltpu.async_copy(ylo_hbm.at[srci.at[j]], rows[b], gsem[b])

        @pl.when(c == 1)
        def _():
            pltpu.async_copy(yhi_hbm.at[srci.at[j]], rows[b], gsem[b])

    def _gather_wait(j, b):
        @pl.when(c == 0)
        def _():
            pltpu.make_async_copy(ylo_hbm.at[srci.at[j]], rows[b], gsem[b]).wait()

        @pl.when(c == 1)
        def _():
            pltpu.make_async_copy(yhi_hbm.at[srci.at[j]], rows[b], gsem[b]).wait()

    for j0 in range(PREF):
        _gather(j0, j0)

    def body(j2, carry):
        for b in range(NBUFS):
            j = j2 * NBUFS + b
            _gather_wait(j, b)
            pltpu.async_copy(rows[b], acc.at[dsti.at[j]], ssem[b], add=True)
            jn = j + PREF
            bn = (b + PREF) % NBUFS

            @pl.when(jn < NBLK)
            def _():
                @pl.when(jn - NBUFS >= 0)
                def _():
                    pltpu.make_async_copy(
                        rows[bn], acc.at[dsti.at[jn - NBUFS]], ssem[bn]
                    ).wait()

                _gather(jn, bn)
        return carry

    lax.fori_loop(0, NBLK // NBUFS, body, 0)
    for b0 in range(NBUFS):
        jd = NBLK - NBUFS + b0
        pltpu.make_async_copy(rows[b0], acc.at[dsti.at[jd]], ssem[b0]).wait()
    plsc.subcore_barrier()
    pltpu.sync_copy(acc.at[pl.ds(s * TPB, TPB)], acc_out.at[c, pl.ds(s * TPB, TPB)])


def _make_deg():
    return pl.kernel(
        _deg_body,
        out_type=jax.ShapeDtypeStruct((NC, NPAD, DEGW), jnp.float32),
        mesh=_mesh(),
        compiler_params=pltpu.CompilerParams(use_tc_tiling_on_sc=False),
        scratch_types=[
            pltpu.VMEM((NBLK, K), jnp.int32),
            pltpu.VMEM((K, DEGW), jnp.float32),
            pltpu.VMEM_SHARED((NPAD, DEGW), jnp.float32),
        ],
    )


def _make_gs(hd):
    return pl.kernel(
        _gs_body,
        out_type=jax.ShapeDtypeStruct((NC, NPAD, hd), jnp.float32),
        mesh=_mesh(),
        compiler_params=pltpu.CompilerParams(use_tc_tiling_on_sc=False),
        scratch_types=[
            pltpu.VMEM((NBLK, K), jnp.int32),
            pltpu.VMEM((NBLK, K), jnp.int32),
        ]
        + [pltpu.VMEM((K, hd), jnp.float32) for _ in range(NBUFS)]
        + [pltpu.VMEM_SHARED((NPAD, hd), jnp.float32)]
        + [pltpu.SemaphoreType.DMA for _ in range(2 * NBUFS)],
    )


def _lin1_body(x_ref, degp_ref, w_ref, y_ref, dis_ref):
    deg = degp_ref[0, :, 0:1] + degp_ref[1, :, 0:1] + 1.0
    dis = lax.rsqrt(deg)
    xw = jnp.dot(x_ref[...], w_ref[...], preferred_element_type=jnp.float32)
    y_ref[...] = xw * dis
    dis_ref[...] = dis


def _lin2_body(acc_ref, y1_ref, dis_ref, w_ref, b_ref, y2_ref):
    dis = dis_ref[...]
    agg = jnp.concatenate([acc_ref[0], acc_ref[1]], axis=-1)
    pre = (agg + y1_ref[...]) * dis + b_ref[...]
    h = jnp.maximum(pre, 0.0)
    y2_ref[...] = jnp.dot(h, w_ref[...], preferred_element_type=jnp.float32) * dis


def _final_body(acc_ref, y2_ref, dis_ref, b_ref, out_ref):
    agg = jnp.concatenate([acc_ref[0], acc_ref[1]], axis=-1)
    out_ref[...] = (agg + y2_ref[...]) * dis_ref[...] + b_ref[...]


def _lin1(x, degp, w1):
    g = NPAD // ROWS_B
    return pl.pallas_call(
        _lin1_body,
        grid=(g,),
        in_specs=[
            pl.BlockSpec((ROWS_B, D_IN), lambda i: (i, 0)),
            pl.BlockSpec((NC, ROWS_B, DEGW), lambda i: (0, i, 0)),
            pl.BlockSpec((D_IN, D_H), lambda i: (0, 0)),
        ],
        out_specs=[
            pl.BlockSpec((ROWS_B, D_H), lambda i: (i, 0)),
            pl.BlockSpec((ROWS_B, 1), lambda i: (i, 0)),
        ],
        out_shape=[
            jax.ShapeDtypeStruct((NPAD, D_H), jnp.float32),
            jax.ShapeDtypeStruct((NPAD, 1), jnp.float32),
        ],
    )(x, degp, w1)


def _lin2(acc1, y1, dis, w2, b1):
    g = NPAD // ROWS_B
    return pl.pallas_call(
        _lin2_body,
        grid=(g,),
        in_specs=[
            pl.BlockSpec((NC, ROWS_B, HD1), lambda i: (0, i, 0)),
            pl.BlockSpec((ROWS_B, D_H), lambda i: (i, 0)),
            pl.BlockSpec((ROWS_B, 1), lambda i: (i, 0)),
            pl.BlockSpec((D_H, D_OUT), lambda i: (0, 0)),
            pl.BlockSpec((1, D_H), lambda i: (0, 0)),
        ],
        out_specs=pl.BlockSpec((ROWS_B, D_OUT), lambda i: (i, 0)),
        out_shape=jax.ShapeDtypeStruct((NPAD, D_OUT), jnp.float32),
    )(acc1, y1, dis, w2, b1)


def _final(acc2, y2, dis, b2):
    g = N // ROWS_F
    return pl.pallas_call(
        _final_body,
        grid=(g,),
        in_specs=[
            pl.BlockSpec((NC, ROWS_F, HD2), lambda i: (0, i, 0)),
            pl.BlockSpec((ROWS_F, D_OUT), lambda i: (i, 0)),
            pl.BlockSpec((ROWS_F, 1), lambda i: (i, 0)),
            pl.BlockSpec((1, D_OUT), lambda i: (0, 0)),
        ],
        out_specs=pl.BlockSpec((ROWS_F, D_OUT), lambda i: (i, 0)),
        out_shape=jax.ShapeDtypeStruct((N, D_OUT), jnp.float32),
    )(acc2, y2, dis, b2)


def kernel(x, edge_index, W1, b1, W2, b2):
    ei = edge_index.astype(jnp.int32)
    pad = jnp.full((2, EPAD - E), NPAD - 1, jnp.int32)
    ei = jnp.concatenate([ei, pad], axis=1)
    src = ei[0].reshape(NS, NBLK, K)
    dst = ei[1].reshape(NS, NBLK, K)
    x = jnp.pad(x, ((0, NPAD - N), (0, 0)))
    zeros_1 = jnp.zeros((NPAD, HD1), jnp.float32)
    zeros_2 = jnp.zeros((NPAD, HD2), jnp.float32)
    zeros_w = jnp.zeros((NPAD, DEGW), jnp.float32)
    ones_w = jnp.ones((K, DEGW), jnp.float32)

    degp = _make_deg()(dst, ones_w, zeros_w)
    y1, dis = _lin1(x, degp, W1)
    acc1 = _make_gs(HD1)(y1[:, :HD1], y1[:, HD1:], src, dst, zeros_1)
    y2 = _lin2(acc1, y1, dis, W2, b1.reshape(1, D_H))
    acc2 = _make_gs(HD2)(y2[:, :HD2], y2[:, HD2:], src, dst, zeros_2)
    out = _final(acc2, y2, dis, b2.reshape(1, D_OUT))
    return out


# speedup vs baseline: 1.5414x; 1.5414x over previous
import jax
import jax.numpy as jnp
from jax import lax
from jax.experimental import pallas as pl
from jax.experimental.pallas import tpu as pltpu
from jax.experimental.pallas import tpu_sc as plsc

N = 10000
E = 320000
D_IN = 128
D_H = 128
D_OUT = 64
HD1 = D_H // 2
HD2 = D_OUT // 2

NC, NS = 2, 16
NPAD = 10240
TPB = NPAD // NS
K = 80
EPAD = 320000
EPT = EPAD // NS
NBLK = EPT // K
DEGW = 16
ROWS_B = 640
ROWS_F = 1000


def _mesh():
    return plsc.VectorSubcoreMesh(
        core_axis_name="c", subcore_axis_name="s", num_cores=NC, num_subcores=NS
    )


def _deg_body(dst_hbm, ones_hbm, zeros_hbm, deg_out, idx_v, ones_v, acc):
    c = lax.axis_index("c")
    s = lax.axis_index("s")
    pltpu.sync_copy(zeros_hbm.at[pl.ds(s * TPB, TPB)], acc.at[pl.ds(s * TPB, TPB)])
    pltpu.sync_copy(dst_hbm.at[s], idx_v)
    pltpu.sync_copy(ones_hbm, ones_v)
    plsc.subcore_barrier()

    def body(j, carry):
        pltpu.sync_copy(ones_v, acc.at[idx_v.at[j]], add=True)
        return carry

    half = NBLK // 2
    lax.fori_loop(c * half, (c + 1) * half, body, 0)
    plsc.subcore_barrier()
    pltpu.sync_copy(acc.at[pl.ds(s * TPB, TPB)], deg_out.at[c, pl.ds(s * TPB, TPB)])


NBUFS = 5
PREF = 2


def _gs_body(ylo_hbm, yhi_hbm, src_hbm, dst_hbm, zeros_hbm, acc_out,
             srci, dsti, rows0, rows1, rows2, rows3, rows4, acc,
             g0, g1, g2, g3, g4, s0, s1, s2, s3, s4):
    c = lax.axis_index("c")
    s = lax.axis_index("s")
    rows = (rows0, rows1, rows2, rows3, rows4)
    gsem = (g0, g1, g2, g3, g4)
    ssem = (s0, s1, s2, s3, s4)
    pltpu.sync_copy(zeros_hbm.at[pl.ds(s * TPB, TPB)], acc.at[pl.ds(s * TPB, TPB)])
    pltpu.sync_copy(src_hbm.at[s], srci)
    pltpu.sync_copy(dst_hbm.at[s], dsti)
    plsc.subcore_barrier()

    def _gather(j, b):
        @pl.when(c == 0)
        def _():
            pltpu.async_copy(ylo_hbm.at[srci.at[j]], rows[b], gsem[b])

        @pl.when(c == 1)
        def _():
            pltpu.async_copy(yhi_hbm.at[srci.at[j]], rows[b], gsem[b])

    def _gather_wait(j, b):
        @pl.when(c == 0)
        def _():
            pltpu.make_async_copy(ylo_hbm.at[srci.at[j]], rows[b], gsem[b]).wait()

        @pl.when(c == 1)
        def _():
            pltpu.make_async_copy(yhi_hbm.at[srci.at[j]], rows[b], gsem[b]).wait()

    for j0 in range(PREF):
        _gather(j0, j0)

    def body(j2, carry):
        for b in range(NBUFS):
            j = j2 * NBUFS + b
            _gather_wait(j, b)
            pltpu.async_copy(rows[b], acc.at[dsti.at[j]], ssem[b], add=True)
            jn = j + PREF
            bn = (b + PREF) % NBUFS

            @pl.when(jn < NBLK)
            def _():
                @pl.when(jn - NBUFS >= 0)
                def _():
                    pltpu.make_async_copy(
                        rows[bn], acc.at[dsti.at[jn - NBUFS]], ssem[bn]
                    ).wait()

                _gather(jn, bn)
        return carry

    lax.fori_loop(0, NBLK // NBUFS, body, 0)
    for b0 in range(NBUFS):
        jd = NBLK - NBUFS + b0
        pltpu.make_async_copy(rows[b0], acc.at[dsti.at[jd]], ssem[b0]).wait()
    plsc.subcore_barrier()
    pltpu.sync_copy(acc.at[pl.ds(s * TPB, TPB)], acc_out.at[c, pl.ds(s * TPB, TPB)])


def _make_deg():
    return pl.kernel(
        _deg_body,
        out_type=jax.ShapeDtypeStruct((NC, NPAD, DEGW), jnp.float32),
        mesh=_mesh(),
        compiler_params=pltpu.CompilerParams(use_tc_tiling_on_sc=False),
        scratch_types=[
            pltpu.VMEM((NBLK, K), jnp.int32),
            pltpu.VMEM((K, DEGW), jnp.float32),
            pltpu.VMEM_SHARED((NPAD, DEGW), jnp.float32),
        ],
    )


def _make_gs(hd):
    return pl.kernel(
        _gs_body,
        out_type=jax.ShapeDtypeStruct((NC, NPAD, hd), jnp.float32),
        mesh=_mesh(),
        compiler_params=pltpu.CompilerParams(use_tc_tiling_on_sc=False),
        scratch_types=[
            pltpu.VMEM((NBLK, K), jnp.int32),
            pltpu.VMEM((NBLK, K), jnp.int32),
        ]
        + [pltpu.VMEM((K, hd), jnp.float32) for _ in range(NBUFS)]
        + [pltpu.VMEM_SHARED((NPAD, hd), jnp.float32)]
        + [pltpu.SemaphoreType.DMA for _ in range(2 * NBUFS)],
    )


def _lin1_body(x_ref, degp_ref, w_ref, y_ref, dis_ref):
    deg = degp_ref[0, :, 0:1] + degp_ref[1, :, 0:1] + 1.0
    dis = lax.rsqrt(deg)
    xw = jnp.dot(x_ref[...], w_ref[...], preferred_element_type=jnp.float32)
    y_ref[...] = xw * dis
    dis_ref[...] = dis


def _lin2_body(acc_ref, y1_ref, dis_ref, w_ref, b_ref, y2_ref):
    dis = dis_ref[...]
    agg = jnp.concatenate([acc_ref[0], acc_ref[1]], axis=-1)
    pre = (agg + y1_ref[...]) * dis + b_ref[...]
    h = jnp.maximum(pre, 0.0)
    y2_ref[...] = jnp.dot(h, w_ref[...], preferred_element_type=jnp.float32) * dis


def _final_body(acc_ref, y2_ref, dis_ref, b_ref, out_ref):
    agg = jnp.concatenate([acc_ref[0], acc_ref[1]], axis=-1)
    out_ref[...] = (agg + y2_ref[...]) * dis_ref[...] + b_ref[...]


def _lin1(x, degp, w1):
    g = NPAD // ROWS_B
    return pl.pallas_call(
        _lin1_body,
        grid=(g,),
        in_specs=[
            pl.BlockSpec((ROWS_B, D_IN), lambda i: (i, 0)),
            pl.BlockSpec((NC, ROWS_B, DEGW), lambda i: (0, i, 0)),
            pl.BlockSpec((D_IN, D_H), lambda i: (0, 0)),
        ],
        out_specs=[
            pl.BlockSpec((ROWS_B, D_H), lambda i: (i, 0)),
            pl.BlockSpec((ROWS_B, 1), lambda i: (i, 0)),
        ],
        out_shape=[
            jax.ShapeDtypeStruct((NPAD, D_H), jnp.float32),
            jax.ShapeDtypeStruct((NPAD, 1), jnp.float32),
        ],
    )(x, degp, w1)


def _lin2(acc1, y1, dis, w2, b1):
    g = NPAD // ROWS_B
    return pl.pallas_call(
        _lin2_body,
        grid=(g,),
        in_specs=[
            pl.BlockSpec((NC, ROWS_B, HD1), lambda i: (0, i, 0)),
            pl.BlockSpec((ROWS_B, D_H), lambda i: (i, 0)),
            pl.BlockSpec((ROWS_B, 1), lambda i: (i, 0)),
            pl.BlockSpec((D_H, D_OUT), lambda i: (0, 0)),
            pl.BlockSpec((1, D_H), lambda i: (0, 0)),
        ],
        out_specs=pl.BlockSpec((ROWS_B, D_OUT), lambda i: (i, 0)),
        out_shape=jax.ShapeDtypeStruct((NPAD, D_OUT), jnp.float32),
    )(acc1, y1, dis, w2, b1)


def _final(acc2, y2, dis, b2):
    g = N // ROWS_F
    return pl.pallas_call(
        _final_body,
        grid=(g,),
        in_specs=[
            pl.BlockSpec((NC, ROWS_F, HD2), lambda i: (0, i, 0)),
            pl.BlockSpec((ROWS_F, D_OUT), lambda i: (i, 0)),
            pl.BlockSpec((ROWS_F, 1), lambda i: (i, 0)),
            pl.BlockSpec((1, D_OUT), lambda i: (0, 0)),
        ],
        out_specs=pl.BlockSpec((ROWS_F, D_OUT), lambda i: (i, 0)),
        out_shape=jax.ShapeDtypeStruct((N, D_OUT), jnp.float32),
    )(acc2, y2, dis, b2)


def kernel(x, edge_index, W1, b1, W2, b2):
    ei = edge_index.astype(jnp.int32)
    pad = jnp.full((2, EPAD - E), NPAD - 1, jnp.int32)
    ei = jnp.concatenate([ei, pad], axis=1)
    src = ei[0].reshape(NS, NBLK, K)
    dst = ei[1].reshape(NS, NBLK, K)
    x = jnp.pad(x, ((0, NPAD - N), (0, 0)))
    zeros_1 = jnp.zeros((NPAD, HD1), jnp.float32)
    zeros_2 = jnp.zeros((NPAD, HD2), jnp.float32)
    zeros_w = jnp.zeros((NPAD, DEGW), jnp.float32)
    ones_w = jnp.ones((K, DEGW), jnp.float32)

    degp = _make_deg()(dst, ones_w, zeros_w)
    y1, dis = _lin1(x, degp, W1)
    acc1 = _make_gs(HD1)(y1[:, :HD1], y1[:, HD1:], src, dst, zeros_1)
    y2 = _lin2(acc1, y1, dis, W2, b1.reshape(1, D_H))
    acc2 = _make_gs(HD2)(y2[:, :HD2], y2[:, HD2:], src, dst, zeros_2)
    out = _final(acc2, y2, dis, b2.reshape(1, D_OUT))
    return out


# speedup vs baseline: 1.5782x; 1.0239x over previous
import jax
import jax.numpy as jnp
from jax import lax
from jax.experimental import pallas as pl
from jax.experimental.pallas import tpu as pltpu
from jax.experimental.pallas import tpu_sc as plsc

N = 10000
E = 320000
D_IN = 128
D_H = 128
D_OUT = 64
HD1 = D_H // 2
HD2 = D_OUT // 2

NC, NS = 2, 16
NPAD = 10240
TPB = NPAD // NS
K = 80
EPAD = 320000
EPT = EPAD // NS
NBLK = EPT // K
DEGW = 16
ROWS_B = 1000


def _mesh():
    return plsc.VectorSubcoreMesh(
        core_axis_name="c", subcore_axis_name="s", num_cores=NC, num_subcores=NS
    )


def _deg_body(dst_hbm, ones_hbm, zeros_hbm, deg_out, idx_v, ones_v, acc):
    c = lax.axis_index("c")
    s = lax.axis_index("s")
    pltpu.sync_copy(zeros_hbm.at[pl.ds(s * TPB, TPB)], acc.at[pl.ds(s * TPB, TPB)])
    pltpu.sync_copy(dst_hbm.at[s], idx_v)
    pltpu.sync_copy(ones_hbm, ones_v)
    plsc.subcore_barrier()

    def body(j, carry):
        pltpu.sync_copy(ones_v, acc.at[idx_v.at[j]], add=True)
        return carry

    half = NBLK // 2
    lax.fori_loop(c * half, (c + 1) * half, body, 0)
    plsc.subcore_barrier()
    pltpu.sync_copy(acc.at[pl.ds(s * TPB, TPB)], deg_out.at[c, pl.ds(s * TPB, TPB)])


NBUFS = 5
PREF = 2


def _gs_body(ylo_hbm, yhi_hbm, src_hbm, dst_hbm, zeros_hbm, acc_out,
             srci, dsti, rows0, rows1, rows2, rows3, rows4, acc,
             g0, g1, g2, g3, g4, s0, s1, s2, s3, s4):
    c = lax.axis_index("c")
    s = lax.axis_index("s")
    rows = (rows0, rows1, rows2, rows3, rows4)
    gsem = (g0, g1, g2, g3, g4)
    ssem = (s0, s1, s2, s3, s4)
    pltpu.sync_copy(zeros_hbm.at[pl.ds(s * TPB, TPB)], acc.at[pl.ds(s * TPB, TPB)])
    pltpu.sync_copy(src_hbm.at[s], srci)
    pltpu.sync_copy(dst_hbm.at[s], dsti)
    plsc.subcore_barrier()

    def _gather(j, b):
        @pl.when(c == 0)
        def _():
            pltpu.async_copy(ylo_hbm.at[srci.at[j]], rows[b], gsem[b])

        @pl.when(c == 1)
        def _():
            pltpu.async_copy(yhi_hbm.at[srci.at[j]], rows[b], gsem[b])

    def _gather_wait(j, b):
        @pl.when(c == 0)
        def _():
            pltpu.make_async_copy(ylo_hbm.at[srci.at[j]], rows[b], gsem[b]).wait()

        @pl.when(c == 1)
        def _():
            pltpu.make_async_copy(yhi_hbm.at[srci.at[j]], rows[b], gsem[b]).wait()

    for j0 in range(PREF):
        _gather(j0, j0)

    def body(j2, carry):
        for b in range(NBUFS):
            j = j2 * NBUFS + b
            _gather_wait(j, b)
            pltpu.async_copy(rows[b], acc.at[dsti.at[j]], ssem[b], add=True)
            jn = j + PREF
            bn = (b + PREF) % NBUFS

            @pl.when(jn < NBLK)
            def _():
                @pl.when(jn - NBUFS >= 0)
                def _():
                    pltpu.make_async_copy(
                        rows[bn], acc.at[dsti.at[jn - NBUFS]], ssem[bn]
                    ).wait()

                _gather(jn, bn)
        return carry

    lax.fori_loop(0, NBLK // NBUFS, body, 0)
    for b0 in range(NBUFS):
        jd = NBLK - NBUFS + b0
        pltpu.make_async_copy(rows[b0], acc.at[dsti.at[jd]], ssem[b0]).wait()
    plsc.subcore_barrier()
    pltpu.sync_copy(acc.at[pl.ds(s * TPB, TPB)], acc_out.at[c, pl.ds(s * TPB, TPB)])


def _make_deg():
    return pl.kernel(
        _deg_body,
        out_type=jax.ShapeDtypeStruct((NC, NPAD, DEGW), jnp.float32),
        mesh=_mesh(),
        compiler_params=pltpu.CompilerParams(use_tc_tiling_on_sc=False),
        scratch_types=[
            pltpu.VMEM((NBLK, K), jnp.int32),
            pltpu.VMEM((K, DEGW), jnp.float32),
            pltpu.VMEM_SHARED((NPAD, DEGW), jnp.float32),
        ],
    )


def _make_gs(hd):
    return pl.kernel(
        _gs_body,
        out_type=jax.ShapeDtypeStruct((NC, NPAD, hd), jnp.float32),
        mesh=_mesh(),
        compiler_params=pltpu.CompilerParams(use_tc_tiling_on_sc=False),
        scratch_types=[
            pltpu.VMEM((NBLK, K), jnp.int32),
            pltpu.VMEM((NBLK, K), jnp.int32),
        ]
        + [pltpu.VMEM((K, hd), jnp.float32) for _ in range(NBUFS)]
        + [pltpu.VMEM_SHARED((NPAD, hd), jnp.float32)]
        + [pltpu.SemaphoreType.DMA for _ in range(2 * NBUFS)],
    )


def _lin1_body(x_ref, degp_ref, w_ref, y_ref, dis_ref):
    deg = degp_ref[0, :, 0:1] + degp_ref[1, :, 0:1] + 1.0
    dis = lax.rsqrt(deg)
    xw = jnp.dot(x_ref[...], w_ref[...], preferred_element_type=jnp.float32)
    y_ref[...] = xw * dis
    dis_ref[...] = dis


def _lin2_body(acc_ref, y1_ref, dis_ref, w_ref, b_ref, y2_ref):
    dis = dis_ref[...]
    agg = jnp.concatenate([acc_ref[0], acc_ref[1]], axis=-1)
    pre = (agg + y1_ref[...]) * dis + b_ref[...]
    h = jnp.maximum(pre, 0.0)
    y2_ref[...] = jnp.dot(h, w_ref[...], preferred_element_type=jnp.float32) * dis


def _final_body(acc_ref, y2_ref, dis_ref, b_ref, out_ref):
    agg = jnp.concatenate([acc_ref[0], acc_ref[1]], axis=-1)
    out_ref[...] = (agg + y2_ref[...]) * dis_ref[...] + b_ref[...]


def _lin1(x, degp, w1):
    g = N // ROWS_B
    return pl.pallas_call(
        _lin1_body,
        grid=(g,),
        in_specs=[
            pl.BlockSpec((ROWS_B, D_IN), lambda i: (i, 0)),
            pl.BlockSpec((NC, ROWS_B, DEGW), lambda i: (0, i, 0)),
            pl.BlockSpec((D_IN, D_H), lambda i: (0, 0)),
        ],
        out_specs=[
            pl.BlockSpec((ROWS_B, D_H), lambda i: (i, 0)),
            pl.BlockSpec((ROWS_B, 1), lambda i: (i, 0)),
        ],
        out_shape=[
            jax.ShapeDtypeStruct((N, D_H), jnp.float32),
            jax.ShapeDtypeStruct((N, 1), jnp.float32),
        ],
    )(x, degp, w1)


def _lin2(acc1, y1, dis, w2, b1):
    g = N // ROWS_B
    return pl.pallas_call(
        _lin2_body,
        grid=(g,),
        in_specs=[
            pl.BlockSpec((NC, ROWS_B, HD1), lambda i: (0, i, 0)),
            pl.BlockSpec((ROWS_B, D_H), lambda i: (i, 0)),
            pl.BlockSpec((ROWS_B, 1), lambda i: (i, 0)),
            pl.BlockSpec((D_H, D_OUT), lambda i: (0, 0)),
            pl.BlockSpec((1, D_H), lambda i: (0, 0)),
        ],
        out_specs=pl.BlockSpec((ROWS_B, D_OUT), lambda i: (i, 0)),
        out_shape=jax.ShapeDtypeStruct((N, D_OUT), jnp.float32),
    )(acc1, y1, dis, w2, b1)


def _final(acc2, y2, dis, b2):
    g = N // ROWS_B
    return pl.pallas_call(
        _final_body,
        grid=(g,),
        in_specs=[
            pl.BlockSpec((NC, ROWS_B, HD2), lambda i: (0, i, 0)),
            pl.BlockSpec((ROWS_B, D_OUT), lambda i: (i, 0)),
            pl.BlockSpec((ROWS_B, 1), lambda i: (i, 0)),
            pl.BlockSpec((1, D_OUT), lambda i: (0, 0)),
        ],
        out_specs=pl.BlockSpec((ROWS_B, D_OUT), lambda i: (i, 0)),
        out_shape=jax.ShapeDtypeStruct((N, D_OUT), jnp.float32),
    )(acc2, y2, dis, b2)


def kernel(x, edge_index, W1, b1, W2, b2):
    ei = edge_index.astype(jnp.int32)
    src = ei[0].reshape(NS, NBLK, K)
    dst = ei[1].reshape(NS, NBLK, K)
    zeros_1 = jnp.zeros((NPAD, HD1), jnp.float32)
    zeros_2 = jnp.zeros((NPAD, HD2), jnp.float32)
    zeros_w = jnp.zeros((NPAD, DEGW), jnp.float32)
    ones_w = jnp.ones((K, DEGW), jnp.float32)

    degp = _make_deg()(dst, ones_w, zeros_w)
    y1, dis = _lin1(x, degp, W1)
    acc1 = _make_gs(HD1)(y1[:, :HD1], y1[:, HD1:], src, dst, zeros_1)
    y2 = _lin2(acc1, y1, dis, W2, b1.reshape(1, D_H))
    acc2 = _make_gs(HD2)(y2[:, :HD2], y2[:, HD2:], src, dst, zeros_2)
    out = _final(acc2, y2, dis, b2.reshape(1, D_OUT))
    return out


# speedup vs baseline: 1.6004x; 1.0141x over previous
import jax
import jax.numpy as jnp
from jax import lax
from jax.experimental import pallas as pl
from jax.experimental.pallas import tpu as pltpu
from jax.experimental.pallas import tpu_sc as plsc

N = 10000
E = 320000
D_IN = 128
D_H = 128
D_OUT = 64
HD1 = D_H // 2
HD2 = D_OUT // 2

NC, NS = 2, 16
NPAD = 10240
TPB = NPAD // NS
K = 80
EPAD = 320000
EPT = EPAD // NS
NBLK = EPT // K
DEGW = 16
ROWS_B = 1000


def _mesh():
    return plsc.VectorSubcoreMesh(
        core_axis_name="c", subcore_axis_name="s", num_cores=NC, num_subcores=NS
    )


def _deg_body(dst_hbm, ones_hbm, zeros_hbm, deg_out, idx_v, ones_v, acc):
    c = lax.axis_index("c")
    s = lax.axis_index("s")
    pltpu.sync_copy(zeros_hbm.at[pl.ds(s * TPB, TPB)], acc.at[pl.ds(s * TPB, TPB)])
    pltpu.sync_copy(dst_hbm.at[s], idx_v)
    pltpu.sync_copy(ones_hbm, ones_v)
    plsc.subcore_barrier()

    def body(j, carry):
        pltpu.sync_copy(ones_v, acc.at[idx_v.at[j]], add=True)
        return carry

    half = NBLK // 2
    lax.fori_loop(c * half, (c + 1) * half, body, 0)
    plsc.subcore_barrier()
    pltpu.sync_copy(acc.at[pl.ds(s * TPB, TPB)], deg_out.at[c, pl.ds(s * TPB, TPB)])


NBUFS = 5
PREF = 2


def _gs_body(ylo_hbm, yhi_hbm, src_hbm, dst_hbm, zeros_hbm, acc_out,
             srci, dsti, rows0, rows1, rows2, rows3, rows4, acc,
             g0, g1, g2, g3, g4, s0, s1, s2, s3, s4):
    c = lax.axis_index("c")
    s = lax.axis_index("s")
    rows = (rows0, rows1, rows2, rows3, rows4)
    gsem = (g0, g1, g2, g3, g4)
    ssem = (s0, s1, s2, s3, s4)
    pltpu.sync_copy(zeros_hbm.at[pl.ds(s * TPB, TPB)], acc.at[pl.ds(s * TPB, TPB)])
    pltpu.sync_copy(src_hbm.at[s], srci)
    pltpu.sync_copy(dst_hbm.at[s], dsti)
    plsc.subcore_barrier()

    def _gather(j, b):
        @pl.when(c == 0)
        def _():
            pltpu.async_copy(ylo_hbm.at[srci.at[j]], rows[b], gsem[b])

        @pl.when(c == 1)
        def _():
            pltpu.async_copy(yhi_hbm.at[srci.at[j]], rows[b], gsem[b])

    def _gather_wait(j, b):
        @pl.when(c == 0)
        def _():
            pltpu.make_async_copy(ylo_hbm.at[srci.at[j]], rows[b], gsem[b]).wait()

        @pl.when(c == 1)
        def _():
            pltpu.make_async_copy(yhi_hbm.at[srci.at[j]], rows[b], gsem[b]).wait()

    for j0 in range(PREF):
        _gather(j0, j0)

    def body(j2, carry):
        for b in range(NBUFS):
            j = j2 * NBUFS + b
            _gather_wait(j, b)
            pltpu.async_copy(rows[b], acc.at[dsti.at[j]], ssem[b], add=True)
            jn = j + PREF
            bn = (b + PREF) % NBUFS

            @pl.when(jn < NBLK)
            def _():
                @pl.when(jn - NBUFS >= 0)
                def _():
                    pltpu.make_async_copy(
                        rows[bn], acc.at[dsti.at[jn - NBUFS]], ssem[bn]
                    ).wait()

                _gather(jn, bn)
        return carry

    lax.fori_loop(0, NBLK // NBUFS, body, 0)
    for b0 in range(NBUFS):
        jd = NBLK - NBUFS + b0
        pltpu.make_async_copy(rows[b0], acc.at[dsti.at[jd]], ssem[b0]).wait()
    plsc.subcore_barrier()
    pltpu.sync_copy(acc.at[pl.ds(s * TPB, TPB)], acc_out.at[c, pl.ds(s * TPB, TPB)])


def _make_deg():
    return pl.kernel(
        _deg_body,
        out_type=jax.ShapeDtypeStruct((NC, NPAD, DEGW), jnp.float32),
        mesh=_mesh(),
        compiler_params=pltpu.CompilerParams(use_tc_tiling_on_sc=False),
        scratch_types=[
            pltpu.VMEM((NBLK, K), jnp.int32),
            pltpu.VMEM((K, DEGW), jnp.float32),
            pltpu.VMEM_SHARED((NPAD, DEGW), jnp.float32),
        ],
    )


def _make_gs(hd):
    return pl.kernel(
        _gs_body,
        out_type=jax.ShapeDtypeStruct((NC, NPAD, hd), jnp.float32),
        mesh=_mesh(),
        compiler_params=pltpu.CompilerParams(use_tc_tiling_on_sc=False),
        scratch_types=[
            pltpu.VMEM((NBLK, K), jnp.int32),
            pltpu.VMEM((NBLK, K), jnp.int32),
        ]
        + [pltpu.VMEM((K, hd), jnp.float32) for _ in range(NBUFS)]
        + [pltpu.VMEM_SHARED((NPAD, hd), jnp.float32)]
        + [pltpu.SemaphoreType.DMA for _ in range(2 * NBUFS)],
    )


def _lin1_body(x_ref, degp_ref, w_ref, ylo_ref, yhi_ref, dis_ref):
    deg = degp_ref[0, :, 0:1] + degp_ref[1, :, 0:1] + 1.0
    dis = lax.rsqrt(deg)
    xw = jnp.dot(x_ref[...], w_ref[...], preferred_element_type=jnp.float32)
    y = xw * dis
    ylo_ref[...] = y[:, :HD1]
    yhi_ref[...] = y[:, HD1:]
    dis_ref[...] = dis


def _lin2_body(acc_ref, ylo_ref, yhi_ref, dis_ref, w_ref, b_ref, y2lo_ref, y2hi_ref):
    dis = dis_ref[...]
    y1 = jnp.concatenate([ylo_ref[...], yhi_ref[...]], axis=-1)
    agg = jnp.concatenate([acc_ref[0], acc_ref[1]], axis=-1)
    pre = (agg + y1) * dis + b_ref[...]
    h = jnp.maximum(pre, 0.0)
    y2 = jnp.dot(h, w_ref[...], preferred_element_type=jnp.float32) * dis
    y2lo_ref[...] = y2[:, :HD2]
    y2hi_ref[...] = y2[:, HD2:]


def _final_body(acc_ref, y2lo_ref, y2hi_ref, dis_ref, b_ref, out_ref):
    y2 = jnp.concatenate([y2lo_ref[...], y2hi_ref[...]], axis=-1)
    agg = jnp.concatenate([acc_ref[0], acc_ref[1]], axis=-1)
    out_ref[...] = (agg + y2) * dis_ref[...] + b_ref[...]


def _lin1(x, degp, w1):
    g = N // ROWS_B
    return pl.pallas_call(
        _lin1_body,
        grid=(g,),
        in_specs=[
            pl.BlockSpec((ROWS_B, D_IN), lambda i: (i, 0)),
            pl.BlockSpec((NC, ROWS_B, DEGW), lambda i: (0, i, 0)),
            pl.BlockSpec((D_IN, D_H), lambda i: (0, 0)),
        ],
        out_specs=[
            pl.BlockSpec((ROWS_B, HD1), lambda i: (i, 0)),
            pl.BlockSpec((ROWS_B, HD1), lambda i: (i, 0)),
            pl.BlockSpec((ROWS_B, 1), lambda i: (i, 0)),
        ],
        out_shape=[
            jax.ShapeDtypeStruct((N, HD1), jnp.float32),
            jax.ShapeDtypeStruct((N, HD1), jnp.float32),
            jax.ShapeDtypeStruct((N, 1), jnp.float32),
        ],
    )(x, degp, w1)


def _lin2(acc1, y1lo, y1hi, dis, w2, b1):
    g = N // ROWS_B
    return pl.pallas_call(
        _lin2_body,
        grid=(g,),
        in_specs=[
            pl.BlockSpec((NC, ROWS_B, HD1), lambda i: (0, i, 0)),
            pl.BlockSpec((ROWS_B, HD1), lambda i: (i, 0)),
            pl.BlockSpec((ROWS_B, HD1), lambda i: (i, 0)),
            pl.BlockSpec((ROWS_B, 1), lambda i: (i, 0)),
            pl.BlockSpec((D_H, D_OUT), lambda i: (0, 0)),
            pl.BlockSpec((1, D_H), lambda i: (0, 0)),
        ],
        out_specs=[
            pl.BlockSpec((ROWS_B, HD2), lambda i: (i, 0)),
            pl.BlockSpec((ROWS_B, HD2), lambda i: (i, 0)),
        ],
        out_shape=[
            jax.ShapeDtypeStruct((N, HD2), jnp.float32),
            jax.ShapeDtypeStruct((N, HD2), jnp.float32),
        ],
    )(acc1, y1lo, y1hi, dis, w2, b1)


def _final(acc2, y2lo, y2hi, dis, b2):
    g = N // ROWS_B
    return pl.pallas_call(
        _final_body,
        grid=(g,),
        in_specs=[
            pl.BlockSpec((NC, ROWS_B, HD2), lambda i: (0, i, 0)),
            pl.BlockSpec((ROWS_B, HD2), lambda i: (i, 0)),
            pl.BlockSpec((ROWS_B, HD2), lambda i: (i, 0)),
            pl.BlockSpec((ROWS_B, 1), lambda i: (i, 0)),
            pl.BlockSpec((1, D_OUT), lambda i: (0, 0)),
        ],
        out_specs=pl.BlockSpec((ROWS_B, D_OUT), lambda i: (i, 0)),
        out_shape=jax.ShapeDtypeStruct((N, D_OUT), jnp.float32),
    )(acc2, y2lo, y2hi, dis, b2)


def kernel(x, edge_index, W1, b1, W2, b2):
    ei = edge_index.astype(jnp.int32)
    src = ei[0].reshape(NS, NBLK, K)
    dst = ei[1].reshape(NS, NBLK, K)
    zeros_1 = jnp.zeros((NPAD, HD1), jnp.float32)
    zeros_2 = jnp.zeros((NPAD, HD2), jnp.float32)
    zeros_w = jnp.zeros((NPAD, DEGW), jnp.float32)
    ones_w = jnp.ones((K, DEGW), jnp.float32)

    degp = _make_deg()(dst, ones_w, zeros_w)
    y1lo, y1hi, dis = _lin1(x, degp, W1)
    acc1 = _make_gs(HD1)(y1lo, y1hi, src, dst, zeros_1)
    y2lo, y2hi = _lin2(acc1, y1lo, y1hi, dis, W2, b1.reshape(1, D_H))
    acc2 = _make_gs(HD2)(y2lo, y2hi, src, dst, zeros_2)
    out = _final(acc2, y2lo, y2hi, dis, b2.reshape(1, D_OUT))
    return out


# speedup vs baseline: 1.8849x; 1.1778x over previous
import jax
import jax.numpy as jnp
from jax import lax
from jax.experimental import pallas as pl
from jax.experimental.pallas import tpu as pltpu
from jax.experimental.pallas import tpu_sc as plsc

N = 10000
E = 320000
D_IN = 128
D_H = 128
D_OUT = 64
HD1 = D_H // 2
HD2 = D_OUT // 2

NC, NS = 2, 16
NPAD = 10240
TPB = NPAD // NS
K = 80
EPAD = 320000
EPT = EPAD // NS
NBLK = EPT // K
DEGW = 16
ROWS_B = 1000


def _mesh():
    return plsc.VectorSubcoreMesh(
        core_axis_name="c", subcore_axis_name="s", num_cores=NC, num_subcores=NS
    )


def _deg_body(dst_hbm, ones_hbm, zeros_hbm, deg_out, idx_v, ones_v, acc):
    c = lax.axis_index("c")
    s = lax.axis_index("s")
    pltpu.sync_copy(zeros_hbm.at[pl.ds(s * TPB, TPB)], acc.at[pl.ds(s * TPB, TPB)])
    pltpu.sync_copy(dst_hbm.at[s], idx_v)
    pltpu.sync_copy(ones_hbm, ones_v)
    plsc.subcore_barrier()

    def body(j, carry):
        pltpu.sync_copy(ones_v, acc.at[idx_v.at[j]], add=True)
        return carry

    half = NBLK // 2
    lax.fori_loop(c * half, (c + 1) * half, body, 0)
    plsc.subcore_barrier()
    pltpu.sync_copy(acc.at[pl.ds(s * TPB, TPB)], deg_out.at[c, pl.ds(s * TPB, TPB)])


NBUFS = 5
PREF = 3


def _gs_body(ylo_hbm, yhi_hbm, src_hbm, dst_hbm, zeros_hbm, acc_out,
             srci, dsti, rows0, rows1, rows2, rows3, rows4, acc,
             g0, g1, g2, g3, g4, s0, s1, s2, s3, s4):
    c = lax.axis_index("c")
    s = lax.axis_index("s")
    rows = (rows0, rows1, rows2, rows3, rows4)
    gsem = (g0, g1, g2, g3, g4)
    ssem = (s0, s1, s2, s3, s4)
    pltpu.sync_copy(zeros_hbm.at[pl.ds(s * TPB, TPB)], acc.at[pl.ds(s * TPB, TPB)])
    pltpu.sync_copy(src_hbm.at[s], srci)
    pltpu.sync_copy(dst_hbm.at[s], dsti)
    plsc.subcore_barrier()

    def _gather(j, b):
        @pl.when(c == 0)
        def _():
            pltpu.async_copy(ylo_hbm.at[srci.at[j]], rows[b], gsem[b])

        @pl.when(c == 1)
        def _():
            pltpu.async_copy(yhi_hbm.at[srci.at[j]], rows[b], gsem[b])

    def _gather_wait(j, b):
        @pl.when(c == 0)
        def _():
            pltpu.make_async_copy(ylo_hbm.at[srci.at[j]], rows[b], gsem[b]).wait()

        @pl.when(c == 1)
        def _():
            pltpu.make_async_copy(yhi_hbm.at[srci.at[j]], rows[b], gsem[b]).wait()

    for j0 in range(PREF):
        _gather(j0, j0)

    def body(j2, carry):
        for b in range(NBUFS):
            j = j2 * NBUFS + b
            _gather_wait(j, b)
            pltpu.async_copy(rows[b], acc.at[dsti.at[j]], ssem[b], add=True)
            jn = j + PREF
            bn = (b + PREF) % NBUFS

            @pl.when(jn < NBLK)
            def _():
                @pl.when(jn - NBUFS >= 0)
                def _():
                    pltpu.make_async_copy(
                        rows[bn], acc.at[dsti.at[jn - NBUFS]], ssem[bn]
                    ).wait()

                _gather(jn, bn)
        return carry

    lax.fori_loop(0, NBLK // NBUFS, body, 0)
    for b0 in range(NBUFS):
        jd = NBLK - NBUFS + b0
        pltpu.make_async_copy(rows[b0], acc.at[dsti.at[jd]], ssem[b0]).wait()
    plsc.subcore_barrier()
    pltpu.sync_copy(acc.at[pl.ds(s * TPB, TPB)], acc_out.at[c, pl.ds(s * TPB, TPB)])


def _make_deg():
    return pl.kernel(
        _deg_body,
        out_type=jax.ShapeDtypeStruct((NC, NPAD, DEGW), jnp.float32),
        mesh=_mesh(),
        compiler_params=pltpu.CompilerParams(use_tc_tiling_on_sc=False),
        scratch_types=[
            pltpu.VMEM((NBLK, K), jnp.int32),
            pltpu.VMEM((K, DEGW), jnp.float32),
            pltpu.VMEM_SHARED((NPAD, DEGW), jnp.float32),
        ],
    )


def _make_gs(hd):
    return pl.kernel(
        _gs_body,
        out_type=jax.ShapeDtypeStruct((NC, NPAD, hd), jnp.float32),
        mesh=_mesh(),
        compiler_params=pltpu.CompilerParams(use_tc_tiling_on_sc=False),
        scratch_types=[
            pltpu.VMEM((NBLK, K), jnp.int32),
            pltpu.VMEM((NBLK, K), jnp.int32),
        ]
        + [pltpu.VMEM((K, hd), jnp.float32) for _ in range(NBUFS)]
        + [pltpu.VMEM_SHARED((NPAD, hd), jnp.float32)]
        + [pltpu.SemaphoreType.DMA for _ in range(2 * NBUFS)],
    )


def _lin1_body(x_ref, degp_ref, w_ref, ylo_ref, yhi_ref, dis_ref):
    deg = degp_ref[0, :, 0:1] + degp_ref[1, :, 0:1] + 1.0
    dis = lax.rsqrt(deg)
    xw = jnp.dot(x_ref[...], w_ref[...], preferred_element_type=jnp.float32)
    y = xw * dis
    ylo_ref[...] = y[:, :HD1]
    yhi_ref[...] = y[:, HD1:]
    dis_ref[...] = dis


def _lin2_body(acc_ref, ylo_ref, yhi_ref, dis_ref, w_ref, b_ref, y2lo_ref, y2hi_ref):
    dis = dis_ref[...]
    y1 = jnp.concatenate([ylo_ref[...], yhi_ref[...]], axis=-1)
    agg = jnp.concatenate([acc_ref[0], acc_ref[1]], axis=-1)
    pre = (agg + y1) * dis + b_ref[...]
    h = jnp.maximum(pre, 0.0)
    y2 = jnp.dot(h, w_ref[...], preferred_element_type=jnp.float32) * dis
    y2lo_ref[...] = y2[:, :HD2]
    y2hi_ref[...] = y2[:, HD2:]


def _final_body(acc_ref, y2lo_ref, y2hi_ref, dis_ref, b_ref, out_ref):
    y2 = jnp.concatenate([y2lo_ref[...], y2hi_ref[...]], axis=-1)
    agg = jnp.concatenate([acc_ref[0], acc_ref[1]], axis=-1)
    out_ref[...] = (agg + y2) * dis_ref[...] + b_ref[...]


def _lin1(x, degp, w1):
    g = N // ROWS_B
    return pl.pallas_call(
        _lin1_body,
        grid=(g,),
        in_specs=[
            pl.BlockSpec((ROWS_B, D_IN), lambda i: (i, 0)),
            pl.BlockSpec((NC, ROWS_B, DEGW), lambda i: (0, i, 0)),
            pl.BlockSpec((D_IN, D_H), lambda i: (0, 0)),
        ],
        out_specs=[
            pl.BlockSpec((ROWS_B, HD1), lambda i: (i, 0)),
            pl.BlockSpec((ROWS_B, HD1), lambda i: (i, 0)),
            pl.BlockSpec((ROWS_B, 1), lambda i: (i, 0)),
        ],
        out_shape=[
            jax.ShapeDtypeStruct((N, HD1), jnp.float32),
            jax.ShapeDtypeStruct((N, HD1), jnp.float32),
            jax.ShapeDtypeStruct((N, 1), jnp.float32),
        ],
    )(x, degp, w1)


def _lin2(acc1, y1lo, y1hi, dis, w2, b1):
    g = N // ROWS_B
    return pl.pallas_call(
        _lin2_body,
        grid=(g,),
        in_specs=[
            pl.BlockSpec((NC, ROWS_B, HD1), lambda i: (0, i, 0)),
            pl.BlockSpec((ROWS_B, HD1), lambda i: (i, 0)),
            pl.BlockSpec((ROWS_B, HD1), lambda i: (i, 0)),
            pl.BlockSpec((ROWS_B, 1), lambda i: (i, 0)),
            pl.BlockSpec((D_H, D_OUT), lambda i: (0, 0)),
            pl.BlockSpec((1, D_H), lambda i: (0, 0)),
        ],
        out_specs=[
            pl.BlockSpec((ROWS_B, HD2), lambda i: (i, 0)),
            pl.BlockSpec((ROWS_B, HD2), lambda i: (i, 0)),
        ],
        out_shape=[
            jax.ShapeDtypeStruct((N, HD2), jnp.float32),
            jax.ShapeDtypeStruct((N, HD2), jnp.float32),
        ],
    )(acc1, y1lo, y1hi, dis, w2, b1)


def _final(acc2, y2lo, y2hi, dis, b2):
    g = N // ROWS_B
    return pl.pallas_call(
        _final_body,
        grid=(g,),
        in_specs=[
            pl.BlockSpec((NC, ROWS_B, HD2), lambda i: (0, i, 0)),
            pl.BlockSpec((ROWS_B, HD2), lambda i: (i, 0)),
            pl.BlockSpec((ROWS_B, HD2), lambda i: (i, 0)),
            pl.BlockSpec((ROWS_B, 1), lambda i: (i, 0)),
            pl.BlockSpec((1, D_OUT), lambda i: (0, 0)),
        ],
        out_specs=pl.BlockSpec((ROWS_B, D_OUT), lambda i: (i, 0)),
        out_shape=jax.ShapeDtypeStruct((N, D_OUT), jnp.float32),
    )(acc2, y2lo, y2hi, dis, b2)


def kernel(x, edge_index, W1, b1, W2, b2):
    ei = edge_index.astype(jnp.int32)
    src = ei[0].reshape(NS, NBLK, K)
    dst = ei[1].reshape(NS, NBLK, K)
    zeros_1 = jnp.zeros((NPAD, HD1), jnp.float32)
    zeros_2 = jnp.zeros((NPAD, HD2), jnp.float32)
    zeros_w = jnp.zeros((NPAD, DEGW), jnp.float32)
    ones_w = jnp.ones((K, DEGW), jnp.float32)

    degp = _make_deg()(dst, ones_w, zeros_w)
    y1lo, y1hi, dis = _lin1(x, degp, W1)
    acc1 = _make_gs(HD1)(y1lo, y1hi, src, dst, zeros_1)
    y2lo, y2hi = _lin2(acc1, y1lo, y1hi, dis, W2, b1.reshape(1, D_H))
    acc2 = _make_gs(HD2)(y2lo, y2hi, src, dst, zeros_2)
    out = _final(acc2, y2lo, y2hi, dis, b2.reshape(1, D_OUT))
    return out


# speedup vs baseline: 2.0235x; 1.0735x over previous
import jax
import jax.numpy as jnp
from jax import lax
from jax.experimental import pallas as pl
from jax.experimental.pallas import tpu as pltpu
from jax.experimental.pallas import tpu_sc as plsc

N = 10000
E = 320000
D_IN = 128
D_H = 128
D_OUT = 64
HD1 = D_H // 2
HD2 = D_OUT // 2

NC, NS = 2, 16
NPAD = 10240
TPB = NPAD // NS
K = 80
EPAD = 320000
EPT = EPAD // NS
NBLK = EPT // K
DEGW = 16
ROWS_B = 1000


def _mesh():
    return plsc.VectorSubcoreMesh(
        core_axis_name="c", subcore_axis_name="s", num_cores=NC, num_subcores=NS
    )


def _deg_body(dst_hbm, ones_hbm, zeros_hbm, deg_out, idx_v, ones_v, acc):
    c = lax.axis_index("c")
    s = lax.axis_index("s")
    pltpu.sync_copy(zeros_hbm.at[pl.ds(s * TPB, TPB)], acc.at[pl.ds(s * TPB, TPB)])
    pltpu.sync_copy(dst_hbm.at[s], idx_v)
    pltpu.sync_copy(ones_hbm, ones_v)
    plsc.subcore_barrier()

    def body(j, carry):
        pltpu.sync_copy(ones_v, acc.at[idx_v.at[j]], add=True)
        return carry

    half = NBLK // 2
    lax.fori_loop(c * half, (c + 1) * half, body, 0)
    plsc.subcore_barrier()
    pltpu.sync_copy(acc.at[pl.ds(s * TPB, TPB)], deg_out.at[c, pl.ds(s * TPB, TPB)])


NBUFS = 5
PREF = 4


def _gs_body(ylo_hbm, yhi_hbm, src_hbm, dst_hbm, zeros_hbm, acc_out,
             srci, dsti, rows0, rows1, rows2, rows3, rows4, acc,
             g0, g1, g2, g3, g4, s0, s1, s2, s3, s4):
    c = lax.axis_index("c")
    s = lax.axis_index("s")
    rows = (rows0, rows1, rows2, rows3, rows4)
    gsem = (g0, g1, g2, g3, g4)
    ssem = (s0, s1, s2, s3, s4)
    pltpu.sync_copy(zeros_hbm.at[pl.ds(s * TPB, TPB)], acc.at[pl.ds(s * TPB, TPB)])
    pltpu.sync_copy(src_hbm.at[s], srci)
    pltpu.sync_copy(dst_hbm.at[s], dsti)
    plsc.subcore_barrier()

    def _gather(j, b):
        @pl.when(c == 0)
        def _():
            pltpu.async_copy(ylo_hbm.at[srci.at[j]], rows[b], gsem[b])

        @pl.when(c == 1)
        def _():
            pltpu.async_copy(yhi_hbm.at[srci.at[j]], rows[b], gsem[b])

    def _gather_wait(j, b):
        @pl.when(c == 0)
        def _():
            pltpu.make_async_copy(ylo_hbm.at[srci.at[j]], rows[b], gsem[b]).wait()

        @pl.when(c == 1)
        def _():
            pltpu.make_async_copy(yhi_hbm.at[srci.at[j]], rows[b], gsem[b]).wait()

    for j0 in range(PREF):
        _gather(j0, j0)

    def body(j2, carry):
        for b in range(NBUFS):
            j = j2 * NBUFS + b
            _gather_wait(j, b)
            pltpu.async_copy(rows[b], acc.at[dsti.at[j]], ssem[b], add=True)
            jn = j + PREF
            bn = (b + PREF) % NBUFS

            @pl.when(jn < NBLK)
            def _():
                @pl.when(jn - NBUFS >= 0)
                def _():
                    pltpu.make_async_copy(
                        rows[bn], acc.at[dsti.at[jn - NBUFS]], ssem[bn]
                    ).wait()

                _gather(jn, bn)
        return carry

    lax.fori_loop(0, NBLK // NBUFS, body, 0)
    for b0 in range(NBUFS):
        jd = NBLK - NBUFS + b0
        pltpu.make_async_copy(rows[b0], acc.at[dsti.at[jd]], ssem[b0]).wait()
    plsc.subcore_barrier()
    pltpu.sync_copy(acc.at[pl.ds(s * TPB, TPB)], acc_out.at[c, pl.ds(s * TPB, TPB)])


def _make_deg():
    return pl.kernel(
        _deg_body,
        out_type=jax.ShapeDtypeStruct((NC, NPAD, DEGW), jnp.float32),
        mesh=_mesh(),
        compiler_params=pltpu.CompilerParams(use_tc_tiling_on_sc=False),
        scratch_types=[
            pltpu.VMEM((NBLK, K), jnp.int32),
            pltpu.VMEM((K, DEGW), jnp.float32),
            pltpu.VMEM_SHARED((NPAD, DEGW), jnp.float32),
        ],
    )


def _make_gs(hd):
    return pl.kernel(
        _gs_body,
        out_type=jax.ShapeDtypeStruct((NC, NPAD, hd), jnp.float32),
        mesh=_mesh(),
        compiler_params=pltpu.CompilerParams(use_tc_tiling_on_sc=False),
        scratch_types=[
            pltpu.VMEM((NBLK, K), jnp.int32),
            pltpu.VMEM((NBLK, K), jnp.int32),
        ]
        + [pltpu.VMEM((K, hd), jnp.float32) for _ in range(NBUFS)]
        + [pltpu.VMEM_SHARED((NPAD, hd), jnp.float32)]
        + [pltpu.SemaphoreType.DMA for _ in range(2 * NBUFS)],
    )


def _lin1_body(x_ref, degp_ref, w_ref, ylo_ref, yhi_ref, dis_ref):
    deg = degp_ref[0, :, 0:1] + degp_ref[1, :, 0:1] + 1.0
    dis = lax.rsqrt(deg)
    xw = jnp.dot(x_ref[...], w_ref[...], preferred_element_type=jnp.float32)
    y = xw * dis
    ylo_ref[...] = y[:, :HD1]
    yhi_ref[...] = y[:, HD1:]
    dis_ref[...] = dis


def _lin2_body(acc_ref, ylo_ref, yhi_ref, dis_ref, w_ref, b_ref, y2lo_ref, y2hi_ref):
    dis = dis_ref[...]
    y1 = jnp.concatenate([ylo_ref[...], yhi_ref[...]], axis=-1)
    agg = jnp.concatenate([acc_ref[0], acc_ref[1]], axis=-1)
    pre = (agg + y1) * dis + b_ref[...]
    h = jnp.maximum(pre, 0.0)
    y2 = jnp.dot(h, w_ref[...], preferred_element_type=jnp.float32) * dis
    y2lo_ref[...] = y2[:, :HD2]
    y2hi_ref[...] = y2[:, HD2:]


def _final_body(acc_ref, y2lo_ref, y2hi_ref, dis_ref, b_ref, out_ref):
    y2 = jnp.concatenate([y2lo_ref[...], y2hi_ref[...]], axis=-1)
    agg = jnp.concatenate([acc_ref[0], acc_ref[1]], axis=-1)
    out_ref[...] = (agg + y2) * dis_ref[...] + b_ref[...]


def _lin1(x, degp, w1):
    g = N // ROWS_B
    return pl.pallas_call(
        _lin1_body,
        grid=(g,),
        in_specs=[
            pl.BlockSpec((ROWS_B, D_IN), lambda i: (i, 0)),
            pl.BlockSpec((NC, ROWS_B, DEGW), lambda i: (0, i, 0)),
            pl.BlockSpec((D_IN, D_H), lambda i: (0, 0)),
        ],
        out_specs=[
            pl.BlockSpec((ROWS_B, HD1), lambda i: (i, 0)),
            pl.BlockSpec((ROWS_B, HD1), lambda i: (i, 0)),
            pl.BlockSpec((ROWS_B, 1), lambda i: (i, 0)),
        ],
        out_shape=[
            jax.ShapeDtypeStruct((N, HD1), jnp.float32),
            jax.ShapeDtypeStruct((N, HD1), jnp.float32),
            jax.ShapeDtypeStruct((N, 1), jnp.float32),
        ],
    )(x, degp, w1)


def _lin2(acc1, y1lo, y1hi, dis, w2, b1):
    g = N // ROWS_B
    return pl.pallas_call(
        _lin2_body,
        grid=(g,),
        in_specs=[
            pl.BlockSpec((NC, ROWS_B, HD1), lambda i: (0, i, 0)),
            pl.BlockSpec((ROWS_B, HD1), lambda i: (i, 0)),
            pl.BlockSpec((ROWS_B, HD1), lambda i: (i, 0)),
            pl.BlockSpec((ROWS_B, 1), lambda i: (i, 0)),
            pl.BlockSpec((D_H, D_OUT), lambda i: (0, 0)),
            pl.BlockSpec((1, D_H), lambda i: (0, 0)),
        ],
        out_specs=[
            pl.BlockSpec((ROWS_B, HD2), lambda i: (i, 0)),
            pl.BlockSpec((ROWS_B, HD2), lambda i: (i, 0)),
        ],
        out_shape=[
            jax.ShapeDtypeStruct((N, HD2), jnp.float32),
            jax.ShapeDtypeStruct((N, HD2), jnp.float32),
        ],
    )(acc1, y1lo, y1hi, dis, w2, b1)


def _final(acc2, y2lo, y2hi, dis, b2):
    g = N // ROWS_B
    return pl.pallas_call(
        _final_body,
        grid=(g,),
        in_specs=[
            pl.BlockSpec((NC, ROWS_B, HD2), lambda i: (0, i, 0)),
            pl.BlockSpec((ROWS_B, HD2), lambda i: (i, 0)),
            pl.BlockSpec((ROWS_B, HD2), lambda i: (i, 0)),
            pl.BlockSpec((ROWS_B, 1), lambda i: (i, 0)),
            pl.BlockSpec((1, D_OUT), lambda i: (0, 0)),
        ],
        out_specs=pl.BlockSpec((ROWS_B, D_OUT), lambda i: (i, 0)),
        out_shape=jax.ShapeDtypeStruct((N, D_OUT), jnp.float32),
    )(acc2, y2lo, y2hi, dis, b2)


def kernel(x, edge_index, W1, b1, W2, b2):
    ei = edge_index.astype(jnp.int32)
    src = ei[0].reshape(NS, NBLK, K)
    dst = ei[1].reshape(NS, NBLK, K)
    zeros_1 = jnp.zeros((NPAD, HD1), jnp.float32)
    zeros_2 = jnp.zeros((NPAD, HD2), jnp.float32)
    zeros_w = jnp.zeros((NPAD, DEGW), jnp.float32)
    ones_w = jnp.ones((K, DEGW), jnp.float32)

    degp = _make_deg()(dst, ones_w, zeros_w)
    y1lo, y1hi, dis = _lin1(x, degp, W1)
    acc1 = _make_gs(HD1)(y1lo, y1hi, src, dst, zeros_1)
    y2lo, y2hi = _lin2(acc1, y1lo, y1hi, dis, W2, b1.reshape(1, D_H))
    acc2 = _make_gs(HD2)(y2lo, y2hi, src, dst, zeros_2)
    out = _final(acc2, y2lo, y2hi, dis, b2.reshape(1, D_OUT))
    return out
